# Initial kernel scaffold; baseline (speedup 1.0000x reference)
import jax
import jax.numpy as jnp
from jax.experimental import pallas as pl


def kernel(group_ids, ord_feats, edge_index, edge_attr, pos, pixel_coords, emb, W1, as1, ad1, We1, ae1, b1, W2, as2, ad2, We2, ae2, b2, W3, as3, ad3, We3, ae3, b3, Wh1, bh1, Wh2, bh2):
    raise NotImplementedError("write your pallas kernel here")


# speedup vs baseline: 59.6570x; 59.6570x over previous
import functools

import jax
import jax.numpy as jnp
from jax import lax
from jax.experimental import pallas as pl
from jax.experimental.pallas import tpu as pltpu
from jax.experimental.pallas import tpu_sc as plsc

N = 10000
E = 640000
P = 4096
N_PAD = 10240
HID = 32
ACC_W = 48

NEG = 0.2
R2 = 400.0 * 400.0
INV2S2 = 1.0 / (2.0 * 200.0 * 200.0)

NW = 32
EPW = E // NW
KCH = 80
NCH = EPW // KCH
RPS = N_PAD // 16


def _prep_body(gid_ref, ord_ref, emb_ref, w1_ref, vsd_ref, h_ref, aux_ref):
    gid = gid_ref[...]
    onehot = (gid == lax.broadcasted_iota(jnp.int32, (1, 24), 1)
              ).astype(jnp.float32)
    x16 = jnp.dot(onehot, emb_ref[...], preferred_element_type=jnp.float32)
    x = jnp.concatenate([x16, ord_ref[...]], axis=1)
    h_ref[...] = jnp.dot(x, w1_ref[...], preferred_element_type=jnp.float32)
    aux_ref[...] = jnp.dot(x, vsd_ref[...], preferred_element_type=jnp.float32)


def _ale_body(ea_ref, ve_ref, a1_ref, a2_ref, a3_ref):
    ea = ea_ref[...]
    ve = ve_ref[...]
    a1_ref[...] = jnp.dot(ea, ve[:, 0:2], preferred_element_type=jnp.float32)
    a2_ref[...] = jnp.dot(ea, ve[:, 2:4], preferred_element_type=jnp.float32)
    a3_ref[...] = jnp.dot(ea, ve[:, 4:6], preferred_element_type=jnp.float32)


def _normalize(parts, b):
    u = parts[0] + parts[1]
    den0 = u[:, 32:33] + 1e-16
    den1 = u[:, 33:34] + 1e-16
    xn = jnp.concatenate([u[:, 0:16] / den0, u[:, 16:32] / den1], axis=1)
    return jnp.maximum(xn + b, 0.0)


def _mid_body(parts_ref, b_ref, w_ref, vsd_ref, h_ref, aux_ref):
    xn = _normalize(parts_ref[...], b_ref[...])
    h_ref[...] = jnp.dot(xn, w_ref[...], preferred_element_type=jnp.float32)
    aux_ref[...] = jnp.dot(xn, vsd_ref[...], preferred_element_type=jnp.float32)


def _final_body(parts_ref, b_ref, h_ref):
    h_ref[...] = _normalize(parts_ref[...], b_ref[...])


def _pool_body(pc_ref, posT_ref, pn2_ref, h_ref, wh1_ref, bh1_ref, wh2_ref,
               y_ref, *, bp, nb):
    px = pc_ref[...]
    px2 = (px * px).sum(axis=1, keepdims=True)

    def step(j, carry):
        acc, s = carry
        vT = posT_ref[:, pl.ds(j * nb, nb)]
        pn2 = pn2_ref[:, pl.ds(j * nb, nb)]
        hc = h_ref[pl.ds(j * nb, nb), :]
        cross = lax.dot_general(px, vT, (((1,), (0,)), ((), ())),
                                preferred_element_type=jnp.float32)
        d2 = (px2 + pn2) - 2.0 * cross
        d2 = jnp.maximum(d2, 0.0)
        w = jnp.where(d2 <= R2, jnp.exp(-d2 / 80000.0), 0.0)
        acc = acc + jnp.dot(w, hc, preferred_element_type=jnp.float32)
        s = s + w.sum(axis=1, keepdims=True)
        return acc, s

    acc0 = jnp.zeros((bp, HID), jnp.float32)
    s0 = jnp.zeros((bp, 1), jnp.float32)
    acc, s = lax.fori_loop(0, N_PAD // nb, step, (acc0, s0))
    pooled = jnp.where(s > 0.0, acc / jnp.maximum(s, 1e-30), 0.0)
    hh = jnp.maximum(
        jnp.dot(pooled, wh1_ref[...], preferred_element_type=jnp.float32)
        + bh1_ref[...], 0.0)
    y_ref[...] = jnp.dot(hh, wh2_ref[...], preferred_element_type=jnp.float32)


def _edge_body(src_hbm, dst_hbm, ale_hbm, aux_hbm, h_hbm, parts_hbm,
               aux_v, src_v, dst_v, ale_v, rows_v, msg_v, acc_sp,
               sem, sem_s, sem_d, sem_a):
    c = lax.axis_index("c")
    s = lax.axis_index("s")
    wid = s * 2 + c
    base = wid * EPW

    pltpu.sync_copy(aux_hbm, aux_v)

    zero16 = jnp.zeros((16,), jnp.float32)
    for r in range(KCH):
        for cc in range(ACC_W // 16):
            msg_v[r, pl.ds(cc * 16, 16)] = zero16
    row0 = s * RPS
    for j in range(RPS // KCH):
        pltpu.sync_copy(msg_v, acc_sp.at[pl.ds(row0 + j * KCH, KCH)])
    plsc.subcore_barrier()

    iota16 = lax.iota(jnp.int32, 16)

    def chunk(i, carry):
        eb = base + i * KCH
        cp_src = pltpu.async_copy(src_hbm.at[pl.ds(eb, KCH)], src_v, sem_s)
        cp_dst = pltpu.async_copy(dst_hbm.at[pl.ds(eb, KCH)], dst_v, sem_d)
        cp_ale = pltpu.async_copy(ale_hbm.at[pl.ds(eb, KCH)], ale_v, sem_a)
        cp_src.wait()
        cp_rows = pltpu.async_copy(h_hbm.at[src_v], rows_v, sem)
        cp_dst.wait()
        cp_ale.wait()
        cp_rows.wait()
        for g in range(KCH // 16):
            rowsg = iota16 + g * 16
            src_g = src_v[pl.ds(g * 16, 16)]
            dst_g = dst_v[pl.ds(g * 16, 16)]
            for hh in range(2):
                colh = jnp.full((16,), hh, jnp.int32)
                als = plsc.load_gather(aux_v, [src_g, colh])
                ald = plsc.load_gather(aux_v, [dst_g, colh + 2])
                ale = plsc.load_gather(ale_v, [rowsg, colh])
                a = als + ald + ale
                a = jnp.where(a > 0.0, a, a * NEG)
                exv = jnp.exp(a)
                plsc.store_scatter(msg_v, [rowsg, colh + 32], exv)
                for ch in range(16):
                    colc = jnp.full((16,), hh * 16 + ch, jnp.int32)
                    col = plsc.load_gather(rows_v, [rowsg, colc])
                    plsc.store_scatter(msg_v, [rowsg, colc], col * exv)
        pltpu.sync_copy(msg_v, acc_sp.at[dst_v], add=True)
        return carry

    lax.fori_loop(0, NCH, chunk, 0)
    plsc.subcore_barrier()

    for j in range(RPS // KCH):
        pltpu.sync_copy(acc_sp.at[pl.ds(row0 + j * KCH, KCH)], msg_v)
        pltpu.sync_copy(msg_v, parts_hbm.at[c, pl.ds(row0 + j * KCH, KCH)])


def _make_edge_kernel():
    mesh = plsc.VectorSubcoreMesh(core_axis_name="c", subcore_axis_name="s")
    return pl.kernel(
        _edge_body,
        out_type=jax.ShapeDtypeStruct((2, N_PAD, ACC_W), jnp.float32),
        mesh=mesh,
        compiler_params=pltpu.CompilerParams(needs_layout_passes=False,
                                             use_tc_tiling_on_sc=False),
        scratch_types=[
            pltpu.VMEM((N_PAD, 4), jnp.float32),
            pltpu.VMEM((KCH,), jnp.int32),
            pltpu.VMEM((KCH,), jnp.int32),
            pltpu.VMEM((KCH, 2), jnp.float32),
            pltpu.VMEM((KCH, HID), jnp.float32),
            pltpu.VMEM((KCH, ACC_W), jnp.float32),
            pltpu.VMEM_SHARED((N_PAD, ACC_W), jnp.float32),
            pltpu.SemaphoreType.DMA,
            pltpu.SemaphoreType.DMA,
            pltpu.SemaphoreType.DMA,
            pltpu.SemaphoreType.DMA,
        ],
    )


def _prep_call(gid2, ordf, emb_p, w1, vsd1):
    bn = 2048
    return pl.pallas_call(
        _prep_body,
        grid=(N_PAD // bn,),
        in_specs=[
            pl.BlockSpec((bn, 1), lambda i: (i, 0)),
            pl.BlockSpec((bn, 4), lambda i: (i, 0)),
            pl.BlockSpec((24, 16), lambda i: (0, 0)),
            pl.BlockSpec((20, HID), lambda i: (0, 0)),
            pl.BlockSpec((20, 4), lambda i: (0, 0)),
        ],
        out_specs=[
            pl.BlockSpec((bn, HID), lambda i: (i, 0)),
            pl.BlockSpec((bn, 4), lambda i: (i, 0)),
        ],
        out_shape=[
            jax.ShapeDtypeStruct((N_PAD, HID), jnp.float32),
            jax.ShapeDtypeStruct((N_PAD, 4), jnp.float32),
        ],
    )(gid2, ordf, emb_p, w1, vsd1)


def _ale_call(edge_attr, ve_all):
    be = 12800
    return pl.pallas_call(
        _ale_body,
        grid=(E // be,),
        in_specs=[
            pl.BlockSpec((be, 2), lambda i: (i, 0)),
            pl.BlockSpec((2, 6), lambda i: (0, 0)),
        ],
        out_specs=[pl.BlockSpec((be, 2), lambda i: (i, 0))] * 3,
        out_shape=[jax.ShapeDtypeStruct((E, 2), jnp.float32)] * 3,
    )(edge_attr, ve_all)


def _mid_call(parts, b, w, vsd):
    bn = 2048
    return pl.pallas_call(
        _mid_body,
        grid=(N_PAD // bn,),
        in_specs=[
            pl.BlockSpec((2, bn, ACC_W), lambda i: (0, i, 0)),
            pl.BlockSpec((1, HID), lambda i: (0, 0)),
            pl.BlockSpec((HID, HID), lambda i: (0, 0)),
            pl.BlockSpec((HID, 4), lambda i: (0, 0)),
        ],
        out_specs=[
            pl.BlockSpec((bn, HID), lambda i: (i, 0)),
            pl.BlockSpec((bn, 4), lambda i: (i, 0)),
        ],
        out_shape=[
            jax.ShapeDtypeStruct((N_PAD, HID), jnp.float32),
            jax.ShapeDtypeStruct((N_PAD, 4), jnp.float32),
        ],
    )(parts, b, w, vsd)


def _final_call(parts, b):
    bn = 2048
    return pl.pallas_call(
        _final_body,
        grid=(N_PAD // bn,),
        in_specs=[
            pl.BlockSpec((2, bn, ACC_W), lambda i: (0, i, 0)),
            pl.BlockSpec((1, HID), lambda i: (0, 0)),
        ],
        out_specs=pl.BlockSpec((bn, HID), lambda i: (i, 0)),
        out_shape=jax.ShapeDtypeStruct((N_PAD, HID), jnp.float32),
    )(parts, b)


def _pool_call(pixel_coords, posT, pn2row, h3, wh1, bh1, wh2_p):
    bp = 256
    nb = 1024
    return pl.pallas_call(
        functools.partial(_pool_body, bp=bp, nb=nb),
        grid=(P // bp,),
        in_specs=[
            pl.BlockSpec((bp, 2), lambda i: (i, 0)),
            pl.BlockSpec((2, N_PAD), lambda i: (0, 0)),
            pl.BlockSpec((1, N_PAD), lambda i: (0, 0)),
            pl.BlockSpec((N_PAD, HID), lambda i: (0, 0)),
            pl.BlockSpec((HID, HID), lambda i: (0, 0)),
            pl.BlockSpec((1, HID), lambda i: (0, 0)),
            pl.BlockSpec((HID, 8), lambda i: (0, 0)),
        ],
        out_specs=pl.BlockSpec((bp, 8), lambda i: (i, 0)),
        out_shape=jax.ShapeDtypeStruct((P, 8), jnp.float32),
    )(pixel_coords, posT, pn2row, h3, wh1, bh1, wh2_p)


def _vsd(w, a_s, a_d):
    d = w.shape[0]
    w3 = w.reshape(d, 2, 16)
    vs = (w3 * a_s[None]).sum(-1)
    vd = (w3 * a_d[None]).sum(-1)
    return jnp.concatenate([vs, vd], axis=1)


def _ve(we, a_e):
    return (we.reshape(2, 2, 16) * a_e[None]).sum(-1)


def kernel(group_ids, ord_feats, edge_index, edge_attr, pos, pixel_coords,
           emb, W1, as1, ad1, We1, ae1, b1, W2, as2, ad2, We2, ae2, b2,
           W3, as3, ad3, We3, ae3, b3, Wh1, bh1, Wh2, bh2):
    f32 = jnp.float32
    pad_n = N_PAD - N
    gid2 = jnp.pad(group_ids.astype(jnp.int32), (0, pad_n)).reshape(N_PAD, 1)
    ordf = jnp.pad(ord_feats, ((0, pad_n), (0, 0)))
    pos_p = jnp.pad(pos, ((0, pad_n), (0, 0)), constant_values=1e9)
    emb_p = jnp.pad(emb, ((0, 24 - emb.shape[0]), (0, 0)))

    src = edge_index[0].astype(jnp.int32)
    dst = edge_index[1].astype(jnp.int32)

    vsd1 = _vsd(W1, as1, ad1)
    vsd2 = _vsd(W2, as2, ad2)
    vsd3 = _vsd(W3, as3, ad3)
    ve_all = jnp.concatenate([_ve(We1, ae1), _ve(We2, ae2), _ve(We3, ae3)],
                             axis=1)

    h1, aux1 = _prep_call(gid2, ordf, emb_p, W1, vsd1)
    posT = pos_p.T
    pn2row = (pos_p ** 2).sum(axis=1).reshape(1, N_PAD)
    ale1, ale2, ale3 = _ale_call(edge_attr, ve_all)

    edge_kernel = _make_edge_kernel()
    parts1 = edge_kernel(src, dst, ale1, aux1, h1)
    h2, aux2 = _mid_call(parts1, b1.reshape(1, HID), W2, vsd2)
    parts2 = edge_kernel(src, dst, ale2, aux2, h2)
    h3, aux3 = _mid_call(parts2, b2.reshape(1, HID), W3, vsd3)
    parts3 = edge_kernel(src, dst, ale3, aux3, h3)
    h3f = _final_call(parts3, b3.reshape(1, HID))

    wh2_p = jnp.pad(Wh2, ((0, 0), (0, 7)))
    y8 = _pool_call(pixel_coords, posT, pn2row, h3f, Wh1, bh1.reshape(1, HID),
                    wh2_p)
    return y8[:, 0] + bh2[0]


# speedup vs baseline: 65.3989x; 1.0962x over previous
import functools

import jax
import jax.numpy as jnp
from jax import lax
from jax.experimental import pallas as pl
from jax.experimental.pallas import tpu as pltpu
from jax.experimental.pallas import tpu_sc as plsc

N = 10000
E = 640000
P = 4096
N_PAD = 10240
HID = 32
ACC_W = 48

NEG = 0.2
R2 = 400.0 * 400.0
INV2S2 = 1.0 / (2.0 * 200.0 * 200.0)

NW = 32
EPW = E // NW
KCH = 80
NCH = EPW // KCH
RPS = N_PAD // 16


def _prep_body(gid_ref, ord_ref, emb_ref, w1_ref, vsd_ref, h_ref, aux_ref):
    gid = gid_ref[...]
    onehot = (gid == lax.broadcasted_iota(jnp.int32, (1, 24), 1)
              ).astype(jnp.float32)
    x16 = jnp.dot(onehot, emb_ref[...], preferred_element_type=jnp.float32)
    x = jnp.concatenate([x16, ord_ref[...]], axis=1)
    h_ref[...] = jnp.dot(x, w1_ref[...], preferred_element_type=jnp.float32)
    aux_ref[...] = jnp.dot(x, vsd_ref[...], preferred_element_type=jnp.float32)


def _ale_body(ea_ref, ve_ref, a1_ref, a2_ref, a3_ref):
    ea = ea_ref[...]
    ve = ve_ref[...]
    a1_ref[...] = jnp.dot(ea, ve[:, 0:2], preferred_element_type=jnp.float32)
    a2_ref[...] = jnp.dot(ea, ve[:, 2:4], preferred_element_type=jnp.float32)
    a3_ref[...] = jnp.dot(ea, ve[:, 4:6], preferred_element_type=jnp.float32)


def _normalize(parts, b):
    u = parts[0] + parts[1]
    den0 = u[:, 32:33] + 1e-16
    den1 = u[:, 33:34] + 1e-16
    xn = jnp.concatenate([u[:, 0:16] / den0, u[:, 16:32] / den1], axis=1)
    return jnp.maximum(xn + b, 0.0)


def _mid_body(parts_ref, b_ref, w_ref, vsd_ref, h_ref, aux_ref):
    xn = _normalize(parts_ref[...], b_ref[...])
    h_ref[...] = jnp.dot(xn, w_ref[...], preferred_element_type=jnp.float32)
    aux_ref[...] = jnp.dot(xn, vsd_ref[...], preferred_element_type=jnp.float32)


def _final_body(parts_ref, b_ref, h_ref):
    h_ref[...] = _normalize(parts_ref[...], b_ref[...])


def _pool_body(pc_ref, posT_ref, pn2_ref, h_ref, wh1_ref, bh1_ref, wh2_ref,
               y_ref, *, bp, nb):
    px = pc_ref[...]
    px2 = (px * px).sum(axis=1, keepdims=True)

    def step(j, carry):
        acc, s = carry
        vT = posT_ref[:, pl.ds(j * nb, nb)]
        pn2 = pn2_ref[:, pl.ds(j * nb, nb)]
        hc = h_ref[pl.ds(j * nb, nb), :]
        cross = lax.dot_general(px, vT, (((1,), (0,)), ((), ())),
                                preferred_element_type=jnp.float32)
        d2 = (px2 + pn2) - 2.0 * cross
        d2 = jnp.maximum(d2, 0.0)
        w = jnp.where(d2 <= R2, jnp.exp(-d2 / 80000.0), 0.0)
        acc = acc + jnp.dot(w, hc, preferred_element_type=jnp.float32)
        s = s + w.sum(axis=1, keepdims=True)
        return acc, s

    acc0 = jnp.zeros((bp, HID), jnp.float32)
    s0 = jnp.zeros((bp, 1), jnp.float32)
    acc, s = lax.fori_loop(0, N_PAD // nb, step, (acc0, s0))
    pooled = jnp.where(s > 0.0, acc / jnp.maximum(s, 1e-30), 0.0)
    hh = jnp.maximum(
        jnp.dot(pooled, wh1_ref[...], preferred_element_type=jnp.float32)
        + bh1_ref[...], 0.0)
    y_ref[...] = jnp.dot(hh, wh2_ref[...], preferred_element_type=jnp.float32)


def _edge_body(src_hbm, dst_hbm, ale_hbm, aux_hbm, h_hbm, parts_hbm,
               aux_v, src_v0, src_v1, dst_v0, dst_v1, ale_v0, ale_v1,
               rows_v0, rows_v1, msg_v, acc_sp,
               sem_g0, sem_g1, sem_s0, sem_s1, sem_d0, sem_d1,
               sem_a0, sem_a1):
    c = lax.axis_index("c")
    s = lax.axis_index("s")
    wid = s * 2 + c
    base = wid * EPW
    bufs = ((src_v0, dst_v0, ale_v0, rows_v0, sem_g0, sem_s0, sem_d0, sem_a0),
            (src_v1, dst_v1, ale_v1, rows_v1, sem_g1, sem_s1, sem_d1, sem_a1))

    pltpu.sync_copy(aux_hbm, aux_v)

    zero16 = jnp.zeros((16,), jnp.float32)
    for r in range(KCH):
        for cc in range(ACC_W // 16):
            msg_v[r, pl.ds(cc * 16, 16)] = zero16
    row0 = s * RPS
    for j in range(RPS // KCH):
        pltpu.sync_copy(msg_v, acc_sp.at[pl.ds(row0 + j * KCH, KCH)])
    plsc.subcore_barrier()

    iota16 = lax.iota(jnp.int32, 16)

    def fire_idx(i, p):
        sv, dv, av, _, _, ss, sd, sa = bufs[p]
        eb = base + i * KCH
        pltpu.async_copy(src_hbm.at[pl.ds(eb, KCH)], sv, ss)
        pltpu.async_copy(dst_hbm.at[pl.ds(eb, KCH)], dv, sd)
        pltpu.async_copy(ale_hbm.at[pl.ds(eb, KCH)], av, sa)

    def wait_src(p):
        sv, _, _, _, _, ss, _, _ = bufs[p]
        pltpu.make_async_copy(src_hbm.at[pl.ds(0, KCH)], sv, ss).wait()

    def fire_gather(p):
        sv, _, _, rv, sg, _, _, _ = bufs[p]
        pltpu.async_copy(h_hbm.at[sv], rv, sg)

    def wait_rest(p):
        sv, dv, av, rv, sg, _, sd, sa = bufs[p]
        pltpu.make_async_copy(dst_hbm.at[pl.ds(0, KCH)], dv, sd).wait()
        pltpu.make_async_copy(ale_hbm.at[pl.ds(0, KCH)], av, sa).wait()
        pltpu.make_async_copy(h_hbm.at[sv], rv, sg).wait()

    def compute(p):
        sv, dv, av, rv, _, _, _, _ = bufs[p]
        for g in range(KCH // 16):
            rowsg = iota16 + g * 16
            src_g = sv[pl.ds(g * 16, 16)]
            dst_g = dv[pl.ds(g * 16, 16)]
            for hh in range(2):
                colh = jnp.full((16,), hh, jnp.int32)
                als = plsc.load_gather(aux_v, [src_g, colh])
                ald = plsc.load_gather(aux_v, [dst_g, colh + 2])
                ale = plsc.load_gather(av, [rowsg, colh])
                a = als + ald + ale
                a = jnp.where(a > 0.0, a, a * NEG)
                exv = jnp.exp(a)
                plsc.store_scatter(msg_v, [rowsg, colh + 32], exv)
                for ch in range(16):
                    colc = jnp.full((16,), hh * 16 + ch, jnp.int32)
                    col = plsc.load_gather(rv, [rowsg, colc])
                    plsc.store_scatter(msg_v, [rowsg, colc], col * exv)
        pltpu.sync_copy(msg_v, acc_sp.at[dv], add=True)

    fire_idx(0, 0)
    wait_src(0)
    fire_gather(0)

    def step2(t, carry):
        for p in range(2):
            i = 2 * t + p
            inext = jnp.where(i + 1 >= NCH, 0, i + 1)
            fire_idx(inext, 1 - p)
            wait_rest(p)
            wait_src(1 - p)
            fire_gather(1 - p)
            compute(p)
        return carry

    lax.fori_loop(0, NCH // 2, step2, 0)
    wait_rest(0)
    plsc.subcore_barrier()

    for j in range(RPS // KCH):
        pltpu.sync_copy(acc_sp.at[pl.ds(row0 + j * KCH, KCH)], msg_v)
        pltpu.sync_copy(msg_v, parts_hbm.at[c, pl.ds(row0 + j * KCH, KCH)])


def _make_edge_kernel():
    mesh = plsc.VectorSubcoreMesh(core_axis_name="c", subcore_axis_name="s")
    return pl.kernel(
        _edge_body,
        out_type=jax.ShapeDtypeStruct((2, N_PAD, ACC_W), jnp.float32),
        mesh=mesh,
        compiler_params=pltpu.CompilerParams(needs_layout_passes=False,
                                             use_tc_tiling_on_sc=False),
        scratch_types=[
            pltpu.VMEM((N_PAD, 4), jnp.float32),
            pltpu.VMEM((KCH,), jnp.int32),
            pltpu.VMEM((KCH,), jnp.int32),
            pltpu.VMEM((KCH,), jnp.int32),
            pltpu.VMEM((KCH,), jnp.int32),
            pltpu.VMEM((KCH, 2), jnp.float32),
            pltpu.VMEM((KCH, 2), jnp.float32),
            pltpu.VMEM((KCH, HID), jnp.float32),
            pltpu.VMEM((KCH, HID), jnp.float32),
            pltpu.VMEM((KCH, ACC_W), jnp.float32),
            pltpu.VMEM_SHARED((N_PAD, ACC_W), jnp.float32),
        ] + [pltpu.SemaphoreType.DMA] * 8,
    )


def _prep_call(gid2, ordf, emb_p, w1, vsd1):
    bn = 2048
    return pl.pallas_call(
        _prep_body,
        grid=(N_PAD // bn,),
        in_specs=[
            pl.BlockSpec((bn, 1), lambda i: (i, 0)),
            pl.BlockSpec((bn, 4), lambda i: (i, 0)),
            pl.BlockSpec((24, 16), lambda i: (0, 0)),
            pl.BlockSpec((20, HID), lambda i: (0, 0)),
            pl.BlockSpec((20, 4), lambda i: (0, 0)),
        ],
        out_specs=[
            pl.BlockSpec((bn, HID), lambda i: (i, 0)),
            pl.BlockSpec((bn, 4), lambda i: (i, 0)),
        ],
        out_shape=[
            jax.ShapeDtypeStruct((N_PAD, HID), jnp.float32),
            jax.ShapeDtypeStruct((N_PAD, 4), jnp.float32),
        ],
    )(gid2, ordf, emb_p, w1, vsd1)


def _ale_call(edge_attr, ve_all):
    be = 12800
    return pl.pallas_call(
        _ale_body,
        grid=(E // be,),
        in_specs=[
            pl.BlockSpec((be, 2), lambda i: (i, 0)),
            pl.BlockSpec((2, 6), lambda i: (0, 0)),
        ],
        out_specs=[pl.BlockSpec((be, 2), lambda i: (i, 0))] * 3,
        out_shape=[jax.ShapeDtypeStruct((E, 2), jnp.float32)] * 3,
    )(edge_attr, ve_all)


def _mid_call(parts, b, w, vsd):
    bn = 2048
    return pl.pallas_call(
        _mid_body,
        grid=(N_PAD // bn,),
        in_specs=[
            pl.BlockSpec((2, bn, ACC_W), lambda i: (0, i, 0)),
            pl.BlockSpec((1, HID), lambda i: (0, 0)),
            pl.BlockSpec((HID, HID), lambda i: (0, 0)),
            pl.BlockSpec((HID, 4), lambda i: (0, 0)),
        ],
        out_specs=[
            pl.BlockSpec((bn, HID), lambda i: (i, 0)),
            pl.BlockSpec((bn, 4), lambda i: (i, 0)),
        ],
        out_shape=[
            jax.ShapeDtypeStruct((N_PAD, HID), jnp.float32),
            jax.ShapeDtypeStruct((N_PAD, 4), jnp.float32),
        ],
    )(parts, b, w, vsd)


def _final_call(parts, b):
    bn = 2048
    return pl.pallas_call(
        _final_body,
        grid=(N_PAD // bn,),
        in_specs=[
            pl.BlockSpec((2, bn, ACC_W), lambda i: (0, i, 0)),
            pl.BlockSpec((1, HID), lambda i: (0, 0)),
        ],
        out_specs=pl.BlockSpec((bn, HID), lambda i: (i, 0)),
        out_shape=jax.ShapeDtypeStruct((N_PAD, HID), jnp.float32),
    )(parts, b)


def _pool_call(pixel_coords, posT, pn2row, h3, wh1, bh1, wh2_p):
    bp = 256
    nb = 1024
    return pl.pallas_call(
        functools.partial(_pool_body, bp=bp, nb=nb),
        grid=(P // bp,),
        in_specs=[
            pl.BlockSpec((bp, 2), lambda i: (i, 0)),
            pl.BlockSpec((2, N_PAD), lambda i: (0, 0)),
            pl.BlockSpec((1, N_PAD), lambda i: (0, 0)),
            pl.BlockSpec((N_PAD, HID), lambda i: (0, 0)),
            pl.BlockSpec((HID, HID), lambda i: (0, 0)),
            pl.BlockSpec((1, HID), lambda i: (0, 0)),
            pl.BlockSpec((HID, 8), lambda i: (0, 0)),
        ],
        out_specs=pl.BlockSpec((bp, 8), lambda i: (i, 0)),
        out_shape=jax.ShapeDtypeStruct((P, 8), jnp.float32),
    )(pixel_coords, posT, pn2row, h3, wh1, bh1, wh2_p)


def _vsd(w, a_s, a_d):
    d = w.shape[0]
    w3 = w.reshape(d, 2, 16)
    vs = (w3 * a_s[None]).sum(-1)
    vd = (w3 * a_d[None]).sum(-1)
    return jnp.concatenate([vs, vd], axis=1)


def _ve(we, a_e):
    return (we.reshape(2, 2, 16) * a_e[None]).sum(-1)


def kernel(group_ids, ord_feats, edge_index, edge_attr, pos, pixel_coords,
           emb, W1, as1, ad1, We1, ae1, b1, W2, as2, ad2, We2, ae2, b2,
           W3, as3, ad3, We3, ae3, b3, Wh1, bh1, Wh2, bh2):
    f32 = jnp.float32
    pad_n = N_PAD - N
    gid2 = jnp.pad(group_ids.astype(jnp.int32), (0, pad_n)).reshape(N_PAD, 1)
    ordf = jnp.pad(ord_feats, ((0, pad_n), (0, 0)))
    pos_p = jnp.pad(pos, ((0, pad_n), (0, 0)), constant_values=1e9)
    emb_p = jnp.pad(emb, ((0, 24 - emb.shape[0]), (0, 0)))

    src = edge_index[0].astype(jnp.int32)
    dst = edge_index[1].astype(jnp.int32)

    vsd1 = _vsd(W1, as1, ad1)
    vsd2 = _vsd(W2, as2, ad2)
    vsd3 = _vsd(W3, as3, ad3)
    ve_all = jnp.concatenate([_ve(We1, ae1), _ve(We2, ae2), _ve(We3, ae3)],
                             axis=1)

    h1, aux1 = _prep_call(gid2, ordf, emb_p, W1, vsd1)
    posT = pos_p.T
    pn2row = (pos_p ** 2).sum(axis=1).reshape(1, N_PAD)
    ale1, ale2, ale3 = _ale_call(edge_attr, ve_all)

    edge_kernel = _make_edge_kernel()
    parts1 = edge_kernel(src, dst, ale1, aux1, h1)
    h2, aux2 = _mid_call(parts1, b1.reshape(1, HID), W2, vsd2)
    parts2 = edge_kernel(src, dst, ale2, aux2, h2)
    h3, aux3 = _mid_call(parts2, b2.reshape(1, HID), W3, vsd3)
    parts3 = edge_kernel(src, dst, ale3, aux3, h3)
    h3f = _final_call(parts3, b3.reshape(1, HID))

    wh2_p = jnp.pad(Wh2, ((0, 0), (0, 7)))
    y8 = _pool_call(pixel_coords, posT, pn2row, h3f, Wh1, bh1.reshape(1, HID),
                    wh2_p)
    return y8[:, 0] + bh2[0]
